# Initial kernel scaffold; baseline (speedup 1.0000x reference)
import jax
import jax.numpy as jnp
from jax.experimental import pallas as pl


def kernel(x, W_gate):
    raise NotImplementedError("write your pallas kernel here")


# speedup vs baseline: 5.0585x; 5.0585x over previous
import functools

import jax
import jax.numpy as jnp
from jax.experimental import pallas as pl
from jax.experimental.pallas import tpu as pltpu

_TOP_K = 8


def _router_body(x_ref, w_ref, dense_ref, idx_ref):
    logits = jnp.dot(x_ref[...], w_ref[...], preferred_element_type=jnp.float32)
    B, E = logits.shape
    col = jax.lax.broadcasted_iota(jnp.int32, (B, E), 1)
    work = logits
    acc = jnp.zeros((B, E), jnp.float32)
    ssum = jnp.zeros((B, 1), jnp.float32)
    v0 = None
    idx_cols = []
    for k in range(_TOP_K):
        m = jnp.max(work, axis=1, keepdims=True)
        is_max = work == m
        sel = jnp.min(jnp.where(is_max, col, E), axis=1, keepdims=True)
        one_hot = col == sel
        if k == 0:
            v0 = m
        e = jnp.exp(m - v0)
        acc = acc + jnp.where(one_hot, e, 0.0)
        ssum = ssum + e
        idx_cols.append(sel)
        work = jnp.where(one_hot, -jnp.inf, work)
    dense_ref[...] = acc / ssum
    idx_ref[...] = jnp.concatenate(idx_cols, axis=1)


@functools.partial(jax.jit, static_argnames=("block_t",))
def _router(x, W_gate, block_t=512):
    T, D = x.shape
    E = W_gate.shape[1]
    grid = T // block_t
    return pl.pallas_call(
        _router_body,
        grid=(grid,),
        in_specs=[
            pl.BlockSpec((block_t, D), lambda i: (i, 0)),
            pl.BlockSpec((D, E), lambda i: (0, 0)),
        ],
        out_specs=[
            pl.BlockSpec((block_t, E), lambda i: (i, 0)),
            pl.BlockSpec((block_t, _TOP_K), lambda i: (i, 0)),
        ],
        out_shape=[
            jax.ShapeDtypeStruct((T, E), jnp.float32),
            jax.ShapeDtypeStruct((T, _TOP_K), jnp.int32),
        ],
        compiler_params=pltpu.CompilerParams(
            dimension_semantics=("arbitrary",),
        ),
    )(x, W_gate)


def kernel(x, W_gate):
    dense_gates, topk_idx = _router(x, W_gate)
    return dense_gates, topk_idx


# speedup vs baseline: 5.6274x; 1.1125x over previous
import functools

import jax
import jax.numpy as jnp
from jax.experimental import pallas as pl
from jax.experimental.pallas import tpu as pltpu

_TOP_K = 8


def _router_body(x_ref, w_ref, dense_ref, idx_ref):
    logits = jnp.dot(x_ref[...], w_ref[...], preferred_element_type=jnp.float32)
    B, E = logits.shape
    rcol = jax.lax.broadcasted_iota(jnp.int32, (B, E), 1).astype(jnp.float32)
    rcol = jnp.float32(E - 1) - rcol
    work = logits
    v0 = None
    idx_cols = []
    for k in range(_TOP_K):
        m = jnp.max(work, axis=1, keepdims=True)
        rsel = jnp.max(jnp.where(work == m, rcol, -1.0), axis=1, keepdims=True)
        if k == 0:
            v0 = m
        idx_cols.append(rsel)
        work = jnp.where(rcol == rsel, -jnp.inf, work)
    expall = jnp.where(work == -jnp.inf, jnp.exp(logits - v0), 0.0)
    ssum = jnp.sum(expall, axis=1, keepdims=True)
    dense_ref[...] = expall / ssum
    idx = jnp.float32(E - 1) - jnp.concatenate(idx_cols, axis=1)
    idx_ref[...] = idx.astype(jnp.int32)


@functools.partial(jax.jit, static_argnames=("block_t",))
def _router(x, W_gate, block_t=512):
    T, D = x.shape
    E = W_gate.shape[1]
    grid = T // block_t
    return pl.pallas_call(
        _router_body,
        grid=(grid,),
        in_specs=[
            pl.BlockSpec((block_t, D), lambda i: (i, 0)),
            pl.BlockSpec((D, E), lambda i: (0, 0)),
        ],
        out_specs=[
            pl.BlockSpec((block_t, E), lambda i: (i, 0)),
            pl.BlockSpec((block_t, _TOP_K), lambda i: (i, 0)),
        ],
        out_shape=[
            jax.ShapeDtypeStruct((T, E), jnp.float32),
            jax.ShapeDtypeStruct((T, _TOP_K), jnp.int32),
        ],
        compiler_params=pltpu.CompilerParams(
            dimension_semantics=("arbitrary",),
        ),
    )(x, W_gate)


def kernel(x, W_gate):
    dense_gates, topk_idx = _router(x, W_gate)
    return dense_gates, topk_idx


# speedup vs baseline: 6.4899x; 1.1533x over previous
import functools

import jax
import jax.numpy as jnp
from jax.experimental import pallas as pl
from jax.experimental.pallas import tpu as pltpu

_TOP_K = 8


def _router_body(x_ref, w_ref, dense_ref, idx_ref, lg_ref):
    logits = lg_ref[...]
    B, E = logits.shape
    rcol = jax.lax.broadcasted_iota(jnp.int32, (B, E), 1).astype(jnp.float32)
    rcol = jnp.float32(E - 1) - rcol
    work = logits
    v0 = None
    idx_cols = []
    for k in range(_TOP_K):
        m = jnp.max(work, axis=1, keepdims=True)
        rsel = jnp.max(jnp.where(work == m, rcol, -1.0), axis=1, keepdims=True)
        if k == 0:
            v0 = m
        idx_cols.append(rsel)
        work = jnp.where(rcol == rsel, -jnp.inf, work)
    expall = jnp.where(work == -jnp.inf, jnp.exp(logits - v0), 0.0)
    ssum = jnp.sum(expall, axis=1, keepdims=True)
    dense_ref[...] = expall / ssum
    idx = jnp.float32(E - 1) - jnp.concatenate(idx_cols, axis=1)
    idx_ref[...] = idx.astype(jnp.int32)
    lg_ref[...] = jnp.dot(x_ref[...], w_ref[...],
                          preferred_element_type=jnp.float32)


@functools.partial(jax.jit, static_argnames=("block_t",))
def _router(x, W_gate, block_t=512):
    T, D = x.shape
    E = W_gate.shape[1]
    nb = T // block_t
    return pl.pallas_call(
        _router_body,
        grid=(nb + 1,),
        in_specs=[
            pl.BlockSpec((block_t, D), lambda i: (jnp.minimum(i, nb - 1), 0)),
            pl.BlockSpec((D, E), lambda i: (0, 0)),
        ],
        out_specs=[
            pl.BlockSpec((block_t, E), lambda i: (jnp.maximum(i - 1, 0), 0)),
            pl.BlockSpec((block_t, _TOP_K), lambda i: (jnp.maximum(i - 1, 0), 0)),
        ],
        out_shape=[
            jax.ShapeDtypeStruct((T, E), jnp.float32),
            jax.ShapeDtypeStruct((T, _TOP_K), jnp.int32),
        ],
        scratch_shapes=[pltpu.VMEM((block_t, E), jnp.float32)],
        compiler_params=pltpu.CompilerParams(
            dimension_semantics=("arbitrary",),
        ),
    )(x, W_gate)


def kernel(x, W_gate):
    dense_gates, topk_idx = _router(x, W_gate)
    return dense_gates, topk_idx


# speedup vs baseline: 6.6305x; 1.0217x over previous
import functools

import jax
import jax.numpy as jnp
from jax.experimental import pallas as pl
from jax.experimental.pallas import tpu as pltpu

_TOP_K = 8


def _router_body(x_ref, w_ref, dense_ref, idx_ref, lg_ref):
    logits = lg_ref[...]
    B, E = logits.shape
    rcol = jax.lax.broadcasted_iota(jnp.int32, (B, E), 1).astype(jnp.float32)
    rcol = jnp.float32(E - 1) - rcol
    work = logits
    v0 = None
    idx_cols = []
    for k in range(_TOP_K):
        m = jnp.max(work, axis=1, keepdims=True)
        rsel = jnp.max(jnp.where(work == m, rcol, -1.0), axis=1, keepdims=True)
        if k == 0:
            v0 = m
        idx_cols.append(rsel)
        work = jnp.where(rcol == rsel, -jnp.inf, work)
    expall = jnp.where(work == -jnp.inf, jnp.exp(logits - v0), 0.0)
    ssum = jnp.sum(expall, axis=1, keepdims=True)
    dense_ref[...] = expall / ssum
    idx = jnp.float32(E - 1) - jnp.concatenate(idx_cols, axis=1)
    idx_ref[...] = idx.astype(jnp.int32)
    lg_ref[...] = jnp.dot(x_ref[...], w_ref[...],
                          preferred_element_type=jnp.float32)


@functools.partial(jax.jit, static_argnames=("block_t",))
def _router(x, W_gate, block_t=1024):
    T, D = x.shape
    E = W_gate.shape[1]
    nb = T // block_t
    return pl.pallas_call(
        _router_body,
        grid=(nb + 1,),
        in_specs=[
            pl.BlockSpec((block_t, D), lambda i: (jnp.minimum(i, nb - 1), 0)),
            pl.BlockSpec((D, E), lambda i: (0, 0)),
        ],
        out_specs=[
            pl.BlockSpec((block_t, E), lambda i: (jnp.maximum(i - 1, 0), 0)),
            pl.BlockSpec((block_t, _TOP_K), lambda i: (jnp.maximum(i - 1, 0), 0)),
        ],
        out_shape=[
            jax.ShapeDtypeStruct((T, E), jnp.float32),
            jax.ShapeDtypeStruct((T, _TOP_K), jnp.int32),
        ],
        scratch_shapes=[pltpu.VMEM((block_t, E), jnp.float32)],
        compiler_params=pltpu.CompilerParams(
            dimension_semantics=("arbitrary",),
        ),
    )(x, W_gate)


def kernel(x, W_gate):
    dense_gates, topk_idx = _router(x, W_gate)
    return dense_gates, topk_idx
